# Initial kernel scaffold; baseline (speedup 1.0000x reference)
import jax
import jax.numpy as jnp
from jax.experimental import pallas as pl


def kernel(x):
    raise NotImplementedError("write your pallas kernel here")


# speedup vs baseline: 4.6021x; 4.6021x over previous
import jax
import jax.numpy as jnp
from jax.experimental import pallas as pl
from jax.experimental.pallas import tpu as pltpu

_ANCHOR_W = (116.0, 156.0, 373.0)
_ANCHOR_H = (90.0, 198.0, 326.0)
_A = 3
_CLS = 80
_HB = 8


def _decode_kernel(x_ref, o_ref):
    n, _, hb, w = x_ref.shape
    h0 = (pl.program_id(0) * hb).astype(jnp.float32)
    gx = jax.lax.broadcasted_iota(jnp.int32, (hb, w), 1).astype(jnp.float32)
    gy = jax.lax.broadcasted_iota(jnp.int32, (hb, w), 0).astype(jnp.float32) + h0
    for a in range(_A):
        base = a * (5 + _CLS)
        bx = jax.nn.sigmoid(x_ref[:, base + 0]) + gx[None]
        by = jax.nn.sigmoid(x_ref[:, base + 1]) + gy[None]
        bw = x_ref[:, base + 2] * _ANCHOR_W[a]
        bh = x_ref[:, base + 3] * _ANCHOR_H[a]
        obj = x_ref[:, base + 4]
        logits = x_ref[:, base + 5 : base + 5 + _CLS]
        score = logits * obj[:, None]
        s = score.reshape(n * _CLS, hb, w)
        smax = jnp.max(s, axis=0)
        idx = jax.lax.broadcasted_iota(jnp.int32, (n * _CLS, hb, w), 0).astype(
            jnp.float32
        )
        sarg = jnp.min(
            jnp.where(s == smax[None], idx, jnp.float32(n * _CLS)), axis=0
        )
        o_ref[:, a * 6 + 0] = bx
        o_ref[:, a * 6 + 1] = by
        o_ref[:, a * 6 + 2] = bw
        o_ref[:, a * 6 + 3] = bh
        o_ref[:, a * 6 + 4] = jnp.broadcast_to(smax[None], (n, hb, w))
        o_ref[:, a * 6 + 5] = jnp.broadcast_to(sarg[None], (n, hb, w))


def kernel(x):
    n, c, h, w = x.shape
    return pl.pallas_call(
        _decode_kernel,
        grid=(h // _HB,),
        in_specs=[pl.BlockSpec((n, c, _HB, w), lambda i: (0, 0, i, 0))],
        out_specs=pl.BlockSpec((n, _A * 6, _HB, w), lambda i: (0, 0, i, 0)),
        out_shape=jax.ShapeDtypeStruct((n, _A * 6, h, w), x.dtype),
        compiler_params=pltpu.CompilerParams(
            dimension_semantics=("parallel",),
            vmem_limit_bytes=64 * 1024 * 1024,
        ),
        name="anchor_decode",
    )(x)


# speedup vs baseline: 4.7666x; 1.0357x over previous
import jax
import jax.numpy as jnp
from jax.experimental import pallas as pl
from jax.experimental.pallas import tpu as pltpu

_ANCHOR_W = (116.0, 156.0, 373.0)
_ANCHOR_H = (90.0, 198.0, 326.0)
_A = 3
_CLS = 80
_HB = 16


def _decode_kernel(x_ref, o_ref):
    n, _, hb, w = x_ref.shape
    h0 = (pl.program_id(0) * hb).astype(jnp.float32)
    gx = jax.lax.broadcasted_iota(jnp.int32, (hb, w), 1).astype(jnp.float32)
    gy = jax.lax.broadcasted_iota(jnp.int32, (hb, w), 0).astype(jnp.float32) + h0
    for a in range(_A):
        base = a * (5 + _CLS)
        bx = jax.nn.sigmoid(x_ref[:, base + 0]) + gx[None]
        by = jax.nn.sigmoid(x_ref[:, base + 1]) + gy[None]
        bw = x_ref[:, base + 2] * _ANCHOR_W[a]
        bh = x_ref[:, base + 3] * _ANCHOR_H[a]
        obj = x_ref[:, base + 4]
        logits = x_ref[:, base + 5 : base + 5 + _CLS]
        score = logits * obj[:, None]
        s = score.reshape(n * _CLS, hb, w)
        smax = jnp.max(s, axis=0)
        idx = jax.lax.broadcasted_iota(jnp.int32, (n * _CLS, hb, w), 0).astype(
            jnp.float32
        )
        sarg = jnp.min(
            jnp.where(s == smax[None], idx, jnp.float32(n * _CLS)), axis=0
        )
        o_ref[:, a * 6 + 0] = bx
        o_ref[:, a * 6 + 1] = by
        o_ref[:, a * 6 + 2] = bw
        o_ref[:, a * 6 + 3] = bh
        o_ref[:, a * 6 + 4] = jnp.broadcast_to(smax[None], (n, hb, w))
        o_ref[:, a * 6 + 5] = jnp.broadcast_to(sarg[None], (n, hb, w))


def kernel(x):
    n, c, h, w = x.shape
    return pl.pallas_call(
        _decode_kernel,
        grid=(h // _HB,),
        in_specs=[pl.BlockSpec((n, c, _HB, w), lambda i: (0, 0, i, 0))],
        out_specs=pl.BlockSpec((n, _A * 6, _HB, w), lambda i: (0, 0, i, 0)),
        out_shape=jax.ShapeDtypeStruct((n, _A * 6, h, w), x.dtype),
        compiler_params=pltpu.CompilerParams(
            dimension_semantics=("parallel",),
            vmem_limit_bytes=64 * 1024 * 1024,
        ),
        name="anchor_decode",
    )(x)
